# baseline (speedup 1.0000x reference)
import jax
import jax.numpy as jnp
from jax.experimental import pallas as pl
from jax.experimental.pallas import tpu as pltpu

_C1 = 32
_NB = 8
_SLACK = 36


def _conv_kernel(Wp, x_ref, w1_ref, b1_ref, w2_ref, b2_ref, o_ref,
                 s1_ref, s2_ref):
    nb, P2, Cin = x_ref.shape
    M = nb * P2
    offs = [(dy - 1) * Wp + (dx - 1) for dy in range(3) for dx in range(3)]

    r = jax.lax.broadcasted_iota(jnp.int32, (M, 1), 0) % P2
    hp = r // Wp
    wp = r % Wp
    interior = ((hp >= 1) & (hp <= Wp - 2) & (wp >= 1) & (wp <= Wp - 2))

    s1_ref[0:_SLACK, :] = jnp.zeros((_SLACK, Cin), s1_ref.dtype)
    s1_ref[_SLACK:_SLACK + M, :] = x_ref[...].reshape(M, Cin)
    s1_ref[_SLACK + M:, :] = jnp.zeros((_SLACK, Cin), s1_ref.dtype)
    p1 = jnp.concatenate(
        [s1_ref[_SLACK + o:_SLACK + o + M, :] for o in offs], axis=-1)
    h1 = jnp.dot(p1, w1_ref[...], preferred_element_type=jnp.float32)
    h1 = jnp.where(interior, jnp.maximum(h1 + b1_ref[...], 0.0), 0.0)

    s2_ref[0:_SLACK, :] = jnp.zeros((_SLACK, _C1), s2_ref.dtype)
    s2_ref[_SLACK:_SLACK + M, :] = h1.astype(s2_ref.dtype)
    s2_ref[_SLACK + M:, :] = jnp.zeros((_SLACK, _C1), s2_ref.dtype)
    p2 = jnp.concatenate(
        [s2_ref[_SLACK + o:_SLACK + o + M, :] for o in offs], axis=-1)
    h2 = jnp.dot(p2, w2_ref[...], preferred_element_type=jnp.float32)
    h2 = jnp.maximum(h2 + b2_ref[...], 0.0)

    h2v = h2.reshape(nb, Wp, Wp, _C1)[:, 1:Wp - 1, 1:Wp - 1, :]
    o_ref[...] = h2v.reshape(nb, (Wp - 2) * (Wp - 2), _C1).astype(o_ref.dtype)


def _conv_stack(x_pad_flat, w1, b1, w2, b2):
    import functools
    B, P2, Cin = x_pad_flat.shape
    Wp = int(round(P2 ** 0.5))
    HW = (Wp - 2) * (Wp - 2)
    nb = _NB if B % _NB == 0 else 1
    M = nb * P2
    return pl.pallas_call(
        functools.partial(_conv_kernel, Wp),
        out_shape=jax.ShapeDtypeStruct((B, HW, _C1), jnp.bfloat16),
        grid=(B // nb,),
        in_specs=[
            pl.BlockSpec((nb, P2, Cin), lambda b: (b, 0, 0)),
            pl.BlockSpec((9 * Cin, _C1), lambda b: (0, 0)),
            pl.BlockSpec((1, _C1), lambda b: (0, 0)),
            pl.BlockSpec((9 * _C1, _C1), lambda b: (0, 0)),
            pl.BlockSpec((1, _C1), lambda b: (0, 0)),
        ],
        out_specs=pl.BlockSpec((nb, HW, _C1), lambda b: (b, 0, 0)),
        scratch_shapes=[
            pltpu.VMEM((M + 2 * _SLACK, Cin), jnp.float32),
            pltpu.VMEM((M + 2 * _SLACK, _C1), jnp.float32),
        ],
        compiler_params=pltpu.CompilerParams(
            dimension_semantics=("parallel",)),
    )(x_pad_flat, w1, b1, w2, b2)


def _fc_kernel(x_ref, w_ref, b_ref, o_ref):
    k = pl.program_id(1)
    acc = jnp.dot(x_ref[...], w_ref[...], preferred_element_type=jnp.float32)

    @pl.when(k == 0)
    def _init():
        o_ref[...] = acc + b_ref[...]

    @pl.when(k != 0)
    def _accum():
        o_ref[...] += acc


def _fc(x, w_kn, b_1n):
    B, K = x.shape
    N = w_kn.shape[1]
    bm = B // 2 if B % 2 == 0 else B
    bk = 4096 if K % 4096 == 0 else K
    return pl.pallas_call(
        _fc_kernel,
        out_shape=jax.ShapeDtypeStruct((B, N), jnp.float32),
        grid=(B // bm, K // bk),
        in_specs=[
            pl.BlockSpec((bm, bk), lambda m, k: (m, k)),
            pl.BlockSpec((bk, N), lambda m, k: (k, 0)),
            pl.BlockSpec((1, N), lambda m, k: (0, 0)),
        ],
        out_specs=pl.BlockSpec((bm, N), lambda m, k: (m, 0)),
        compiler_params=pltpu.CompilerParams(
            dimension_semantics=("parallel", "arbitrary")),
    )(x, w_kn, b_1n)


def kernel(w1, b1, w2, b2, fc_w, fc_b, x_nchw):
    if x_nchw.ndim == 3:
        x_nchw = x_nchw[None]
    B = x_nchw.shape[0]
    x = jnp.transpose(x_nchw, (0, 2, 3, 1))
    xp = jnp.pad(x, ((0, 0), (1, 1), (1, 1), (0, 0)))
    xp = xp.reshape(B, xp.shape[1] * xp.shape[2], x.shape[-1])
    h = _conv_stack(xp, w1, b1, w2, b2)
    h = h.reshape(B, -1)
    return _fc(h, fc_w.astype(jnp.bfloat16), fc_b)


# speedup vs baseline: 3.2195x; 3.2195x over previous
import functools

import jax
import jax.numpy as jnp
from jax.experimental import pallas as pl
from jax.experimental.pallas import tpu as pltpu

_C1 = 32
_NB = 8
_L = 1280
_OFF = 36


def _convT_kernel(Wp, x_ref, w1T_ref, b1T_ref, w2T_ref, b2T_ref, o_ref,
                  s2_ref):
    Cin, S = x_ref.shape
    P2 = Wp * Wp
    Sw = S - 2 * _OFF
    offs = [(dy - 1) * Wp + (dx - 1) for dy in range(3) for dx in range(3)]

    p1 = jnp.concatenate(
        [x_ref[:, _OFF + o:_OFF + o + Sw] for o in offs], axis=0)
    h1 = jnp.dot(w1T_ref[...], p1, preferred_element_type=jnp.float32)

    j = jax.lax.broadcasted_iota(jnp.int32, (1, Sw), 1)
    q = (j + _OFF) % _L
    p = q - _OFF
    hp = p // Wp
    wp = p % Wp
    valid = ((p >= 0) & (p < P2) & (hp >= 1) & (hp <= Wp - 2)
             & (wp >= 1) & (wp <= Wp - 2))
    h1 = jnp.where(valid, jnp.maximum(h1 + b1T_ref[...], 0.0), 0.0)

    s2_ref[:, 0:_OFF] = jnp.zeros((_C1, _OFF), s2_ref.dtype)
    s2_ref[:, _OFF:_OFF + Sw] = h1
    s2_ref[:, _OFF + Sw:] = jnp.zeros((_C1, _OFF), s2_ref.dtype)
    p2 = jnp.concatenate(
        [s2_ref[:, _OFF + o:_OFF + o + Sw] for o in offs], axis=0)
    h2 = jnp.dot(w2T_ref[...], p2, preferred_element_type=jnp.float32)
    h2 = jnp.maximum(h2 + b2T_ref[...], 0.0)

    o_ref[:, 0:_OFF] = jnp.zeros((_C1, _OFF), o_ref.dtype)
    o_ref[:, _OFF:_OFF + Sw] = h2.astype(o_ref.dtype)
    o_ref[:, _OFF + Sw:] = jnp.zeros((_C1, _OFF), o_ref.dtype)


def _conv_stack(x_cs, w1T, b1T, w2T, b2T, Wp):
    Cin, BL = x_cs.shape
    B = BL // _L
    nb = _NB if B % _NB == 0 else 1
    S = nb * _L
    return pl.pallas_call(
        functools.partial(_convT_kernel, Wp),
        out_shape=jax.ShapeDtypeStruct((_C1, BL), jnp.bfloat16),
        grid=(B // nb,),
        in_specs=[
            pl.BlockSpec((Cin, S), lambda b: (0, b)),
            pl.BlockSpec((_C1, 9 * Cin), lambda b: (0, 0)),
            pl.BlockSpec((_C1, 1), lambda b: (0, 0)),
            pl.BlockSpec((_C1, 9 * _C1), lambda b: (0, 0)),
            pl.BlockSpec((_C1, 1), lambda b: (0, 0)),
        ],
        out_specs=pl.BlockSpec((_C1, S), lambda b: (0, b)),
        scratch_shapes=[
            pltpu.VMEM((_C1, S), jnp.float32),
        ],
        compiler_params=pltpu.CompilerParams(
            dimension_semantics=("parallel",)),
    )(x_cs, w1T, b1T, w2T, b2T)


def _fcT_kernel(h_ref, w_ref, b_ref, o_ref):
    bm = h_ref.shape[1]
    acc = jnp.broadcast_to(b_ref[...], (bm, b_ref.shape[1])).astype(jnp.float32)
    for c in range(_C1):
        acc = acc + jnp.dot(h_ref[c], w_ref[c],
                            preferred_element_type=jnp.float32)
    o_ref[...] = acc


def _fc(h3, w3, b_1n):
    C, B, L = h3.shape
    N = w3.shape[-1]
    bm = B // 2 if B % 2 == 0 else B
    return pl.pallas_call(
        _fcT_kernel,
        out_shape=jax.ShapeDtypeStruct((B, N), jnp.float32),
        grid=(B // bm,),
        in_specs=[
            pl.BlockSpec((C, bm, L), lambda m: (0, m, 0)),
            pl.BlockSpec((C, L, N), lambda m: (0, 0, 0)),
            pl.BlockSpec((1, N), lambda m: (0, 0)),
        ],
        out_specs=pl.BlockSpec((bm, N), lambda m: (m, 0)),
        compiler_params=pltpu.CompilerParams(
            dimension_semantics=("parallel",)),
    )(h3, w3, b_1n)


def kernel(w1, b1, w2, b2, fc_w, fc_b, x_nchw):
    if x_nchw.ndim == 3:
        x_nchw = x_nchw[None]
    B, Cin, H, W = x_nchw.shape
    Wp = H + 2
    P2 = Wp * Wp

    xp = jnp.pad(x_nchw, ((0, 0), (0, 0), (1, 1), (1, 1)))
    xp = xp.reshape(B, Cin, P2)
    xp = jnp.pad(xp, ((0, 0), (0, 0), (_OFF, _L - _OFF - P2)))
    x_cs = jnp.transpose(xp, (1, 0, 2)).reshape(Cin, B * _L)

    h = _conv_stack(x_cs, w1.T, b1.reshape(_C1, 1), w2.T, b2.reshape(_C1, 1),
                    Wp)
    h3 = h.reshape(_C1, B, _L)

    fw = fc_w.reshape(H * W, _C1, -1)
    N = fw.shape[-1]
    fw = jnp.transpose(fw, (1, 0, 2)).reshape(_C1, H, W, N)
    fw = jnp.pad(fw, ((0, 0), (1, 1), (1, 1), (0, 0))).reshape(_C1, P2, N)
    fw = jnp.pad(fw, ((0, 0), (_OFF, _L - _OFF - P2), (0, 0)))
    return _fc(h3, fw.astype(jnp.bfloat16), fc_b)


# speedup vs baseline: 3.7131x; 1.1533x over previous
import functools

import jax
import jax.numpy as jnp
from jax.experimental import pallas as pl
from jax.experimental.pallas import tpu as pltpu

_C1 = 32
_NB = 16
_L = 1280
_OFF = 36


def _convT_kernel(Wp, x_ref, w1T_ref, b1T_ref, w2T_ref, b2T_ref, o_ref,
                  s2_ref):
    Cin, S = x_ref.shape
    P2 = Wp * Wp
    Sw = S - 2 * _OFF
    offs = [(dy - 1) * Wp + (dx - 1) for dy in range(3) for dx in range(3)]

    p1 = jnp.concatenate(
        [x_ref[:, _OFF + o:_OFF + o + Sw] for o in offs], axis=0)
    h1 = jnp.dot(w1T_ref[...], p1, preferred_element_type=jnp.float32)

    j = jax.lax.broadcasted_iota(jnp.int32, (1, Sw), 1)
    q = (j + _OFF) % _L
    p = q - _OFF
    hp = p // Wp
    wp = p % Wp
    valid = ((p >= 0) & (p < P2) & (hp >= 1) & (hp <= Wp - 2)
             & (wp >= 1) & (wp <= Wp - 2))
    h1 = jnp.where(valid, jnp.maximum(h1 + b1T_ref[...], 0.0), 0.0)

    s2_ref[:, 0:_OFF] = jnp.zeros((_C1, _OFF), s2_ref.dtype)
    s2_ref[:, _OFF:_OFF + Sw] = h1.astype(s2_ref.dtype)
    s2_ref[:, _OFF + Sw:] = jnp.zeros((_C1, _OFF), s2_ref.dtype)
    p2 = jnp.concatenate(
        [s2_ref[:, _OFF + o:_OFF + o + Sw] for o in offs], axis=0)
    h2 = jnp.dot(w2T_ref[...], p2, preferred_element_type=jnp.float32)
    h2 = jnp.maximum(h2 + b2T_ref[...], 0.0)

    o_ref[:, 0:_OFF] = jnp.zeros((_C1, _OFF), o_ref.dtype)
    o_ref[:, _OFF:_OFF + Sw] = h2.astype(o_ref.dtype)
    o_ref[:, _OFF + Sw:] = jnp.zeros((_C1, _OFF), o_ref.dtype)


def _conv_stack(x_cs, w1T, b1T, w2T, b2T, Wp):
    Cin, BL = x_cs.shape
    B = BL // _L
    nb = _NB if B % _NB == 0 else 1
    S = nb * _L
    return pl.pallas_call(
        functools.partial(_convT_kernel, Wp),
        out_shape=jax.ShapeDtypeStruct((_C1, BL), jnp.bfloat16),
        grid=(B // nb,),
        in_specs=[
            pl.BlockSpec((Cin, S), lambda b: (0, b)),
            pl.BlockSpec((_C1, 9 * Cin), lambda b: (0, 0)),
            pl.BlockSpec((_C1, 1), lambda b: (0, 0)),
            pl.BlockSpec((_C1, 9 * _C1), lambda b: (0, 0)),
            pl.BlockSpec((_C1, 1), lambda b: (0, 0)),
        ],
        out_specs=pl.BlockSpec((_C1, S), lambda b: (0, b)),
        scratch_shapes=[
            pltpu.VMEM((_C1, S), jnp.bfloat16),
        ],
        compiler_params=pltpu.CompilerParams(
            dimension_semantics=("parallel",)),
    )(x_cs, w1T, b1T, w2T, b2T)


def _fcT_kernel(h_ref, w_ref, b_ref, o_ref):
    bm = h_ref.shape[1]
    acc = jnp.broadcast_to(b_ref[...], (bm, b_ref.shape[1])).astype(jnp.float32)
    for c in range(_C1):
        acc = acc + jnp.dot(h_ref[c], w_ref[c],
                            preferred_element_type=jnp.float32)
    o_ref[...] = acc


def _fc(h3, w3, b_1n):
    C, B, L = h3.shape
    N = w3.shape[-1]
    bm = B // 2 if B % 2 == 0 else B
    return pl.pallas_call(
        _fcT_kernel,
        out_shape=jax.ShapeDtypeStruct((B, N), jnp.float32),
        grid=(B // bm,),
        in_specs=[
            pl.BlockSpec((C, bm, L), lambda m: (0, m, 0)),
            pl.BlockSpec((C, L, N), lambda m: (0, 0, 0)),
            pl.BlockSpec((1, N), lambda m: (0, 0)),
        ],
        out_specs=pl.BlockSpec((bm, N), lambda m: (m, 0)),
        compiler_params=pltpu.CompilerParams(
            dimension_semantics=("parallel",)),
    )(h3, w3, b_1n)


def kernel(w1, b1, w2, b2, fc_w, fc_b, x_nchw):
    if x_nchw.ndim == 3:
        x_nchw = x_nchw[None]
    B, Cin, H, W = x_nchw.shape
    Wp = H + 2
    P2 = Wp * Wp

    xp = jnp.pad(x_nchw, ((0, 0), (0, 0), (1, 1), (1, 1)))
    xp = xp.reshape(B, Cin, P2)
    xp = jnp.pad(xp, ((0, 0), (0, 0), (_OFF, _L - _OFF - P2)))
    x_cs = jnp.transpose(xp, (1, 0, 2)).reshape(Cin, B * _L)

    h = _conv_stack(x_cs, w1.T, b1.reshape(_C1, 1),
                    w2.T.astype(jnp.bfloat16), b2.reshape(_C1, 1), Wp)
    h3 = h.reshape(_C1, B, _L)

    fw = fc_w.reshape(H * W, _C1, -1)
    N = fw.shape[-1]
    fw = jnp.transpose(fw, (1, 0, 2)).reshape(_C1, H, W, N)
    fw = jnp.pad(fw, ((0, 0), (1, 1), (1, 1), (0, 0))).reshape(_C1, P2, N)
    fw = jnp.pad(fw, ((0, 0), (_OFF, _L - _OFF - P2), (0, 0)))
    return _fc(h3, fw.astype(jnp.bfloat16), fc_b)


# speedup vs baseline: 3.7945x; 1.0219x over previous
import functools

import jax
import jax.numpy as jnp
from jax.experimental import pallas as pl
from jax.experimental.pallas import tpu as pltpu

_C1 = 32
_NB = 16
_L = 1280
_OFF = 36


def _convT_kernel(Wp, x_ref, w1T_ref, b1T_ref, w2T_ref, b2T_ref, o_ref,
                  s2_ref):
    Cin, S = x_ref.shape
    P2 = Wp * Wp
    Sw = S - 2 * _OFF
    offs = [(dy - 1) * Wp + (dx - 1) for dy in range(3) for dx in range(3)]

    p1 = jnp.concatenate(
        [x_ref[:, _OFF + o:_OFF + o + Sw] for o in offs], axis=0)
    h1 = jnp.dot(w1T_ref[...], p1, preferred_element_type=jnp.float32)

    j = jax.lax.broadcasted_iota(jnp.int32, (1, Sw), 1)
    q = (j + _OFF) % _L
    p = q - _OFF
    hp = p // Wp
    wp = p % Wp
    valid = ((p >= 0) & (p < P2) & (hp >= 1) & (hp <= Wp - 2)
             & (wp >= 1) & (wp <= Wp - 2))
    h1 = jnp.where(valid, jnp.maximum(h1 + b1T_ref[...], 0.0), 0.0)

    s2_ref[:, 0:_OFF] = jnp.zeros((_C1, _OFF), s2_ref.dtype)
    s2_ref[:, _OFF:_OFF + Sw] = h1.astype(s2_ref.dtype)
    s2_ref[:, _OFF + Sw:] = jnp.zeros((_C1, _OFF), s2_ref.dtype)
    p2 = jnp.concatenate(
        [s2_ref[:, _OFF + o:_OFF + o + Sw] for o in offs], axis=0)
    h2 = jnp.dot(w2T_ref[...], p2, preferred_element_type=jnp.float32)
    h2 = jnp.maximum(h2 + b2T_ref[...], 0.0)

    nb = S // _L
    o_flat = jnp.concatenate(
        [jnp.zeros((_C1, _OFF), o_ref.dtype), h2.astype(o_ref.dtype),
         jnp.zeros((_C1, _OFF), o_ref.dtype)], axis=1)
    o_ref[...] = o_flat.reshape(_C1, nb, _L)


def _conv_stack(x_cs, w1T, b1T, w2T, b2T, Wp):
    Cin, BL = x_cs.shape
    B = BL // _L
    nb = _NB if B % _NB == 0 else 1
    S = nb * _L
    return pl.pallas_call(
        functools.partial(_convT_kernel, Wp),
        out_shape=jax.ShapeDtypeStruct((_C1, B, _L), jnp.bfloat16),
        grid=(B // nb,),
        in_specs=[
            pl.BlockSpec((Cin, S), lambda b: (0, b)),
            pl.BlockSpec((_C1, 9 * Cin), lambda b: (0, 0)),
            pl.BlockSpec((_C1, 1), lambda b: (0, 0)),
            pl.BlockSpec((_C1, 9 * _C1), lambda b: (0, 0)),
            pl.BlockSpec((_C1, 1), lambda b: (0, 0)),
        ],
        out_specs=pl.BlockSpec((_C1, nb, _L), lambda b: (0, b, 0)),
        scratch_shapes=[
            pltpu.VMEM((_C1, S), jnp.bfloat16),
        ],
        compiler_params=pltpu.CompilerParams(
            dimension_semantics=("parallel",)),
    )(x_cs, w1T, b1T, w2T, b2T)


def _fcT_kernel(h_ref, w_ref, b_ref, o_ref):
    bm = h_ref.shape[1]
    acc = jnp.broadcast_to(b_ref[...], (bm, b_ref.shape[1])).astype(jnp.float32)
    for c in range(_C1):
        acc = acc + jnp.dot(h_ref[c], w_ref[c],
                            preferred_element_type=jnp.float32)
    o_ref[...] = acc


def _fc(h3, w3, b_1n):
    C, B, L = h3.shape
    N = w3.shape[-1]
    bm = B // 2 if B % 2 == 0 else B
    return pl.pallas_call(
        _fcT_kernel,
        out_shape=jax.ShapeDtypeStruct((B, N), jnp.float32),
        grid=(B // bm,),
        in_specs=[
            pl.BlockSpec((C, bm, L), lambda m: (0, m, 0)),
            pl.BlockSpec((C, L, N), lambda m: (0, 0, 0)),
            pl.BlockSpec((1, N), lambda m: (0, 0)),
        ],
        out_specs=pl.BlockSpec((bm, N), lambda m: (m, 0)),
        compiler_params=pltpu.CompilerParams(
            dimension_semantics=("parallel",)),
    )(h3, w3, b_1n)


def kernel(w1, b1, w2, b2, fc_w, fc_b, x_nchw):
    if x_nchw.ndim == 3:
        x_nchw = x_nchw[None]
    B, Cin, H, W = x_nchw.shape
    Wp = H + 2
    P2 = Wp * Wp

    xp = jnp.pad(x_nchw, ((0, 0), (0, 0), (1, 1), (1, 1)))
    xp = xp.reshape(B, Cin, P2)
    xp = jnp.pad(xp, ((0, 0), (0, 0), (_OFF, _L - _OFF - P2)))
    x_cs = jnp.transpose(xp, (1, 0, 2)).reshape(Cin, B * _L)

    h3 = _conv_stack(x_cs, w1.T, b1.reshape(_C1, 1),
                     w2.T.astype(jnp.bfloat16), b2.reshape(_C1, 1), Wp)

    fw = fc_w.astype(jnp.bfloat16).reshape(H * W, _C1, -1)
    N = fw.shape[-1]
    fw = jnp.transpose(fw, (1, 0, 2)).reshape(_C1, H, W, N)
    fw = jnp.pad(fw, ((0, 0), (1, 1), (1, 1), (0, 0))).reshape(_C1, P2, N)
    fw = jnp.pad(fw, ((0, 0), (_OFF, _L - _OFF - P2), (0, 0)))
    return _fc(h3, fw, fc_b)
